# baseline (speedup 1.0000x reference)
import functools

import jax
import jax.numpy as jnp
from jax.experimental import pallas as pl
from jax.experimental.pallas import tpu as pltpu

N_TOPICS = 8
D_OUT = 3


def _dense_body(x_ref, wsel_ref, w1_ref, b1_ref, w2_ref, b2_ref, out_ref,
                topics_ref):
    e = pl.program_id(1)

    @pl.when(e == 0)
    def _gate():
        logits = jnp.dot(x_ref[...], wsel_ref[...],
                         preferred_element_type=jnp.float32)
        best = logits[:, 0:1]
        idx = jnp.zeros((logits.shape[0], 1), jnp.int32)
        for j in range(1, N_TOPICS):
            lj = logits[:, j:j + 1]
            take = lj > best
            best = jnp.where(take, lj, best)
            idx = jnp.where(take, j, idx)
        topics_ref[...] = jnp.broadcast_to(idx, topics_ref.shape)

    h = jnp.maximum(
        jnp.dot(x_ref[...], w1_ref[0], preferred_element_type=jnp.float32)
        + b1_ref[0], 0.0)
    o = jnp.dot(h, w2_ref[0], preferred_element_type=jnp.float32) \
        + b2_ref[0]
    mask = topics_ref[:, :o.shape[1]] == e

    @pl.when(e == 0)
    def _init():
        out_ref[...] = jnp.where(mask, o, 0.0)

    @pl.when(e > 0)
    def _acc():
        out_ref[...] = jnp.where(mask, o, out_ref[...])


def kernel(x, W_sel, W1, b1, W2, b2):
    n_tok, d_in = x.shape
    n_exp, _, d_hid = W1.shape
    d_out = W2.shape[-1]
    tile = 512
    ntiles = n_tok // tile
    b1r = b1.reshape(n_exp, 1, d_hid)
    b2r = b2.reshape(n_exp, 1, d_out)

    out = pl.pallas_call(
        _dense_body,
        grid=(ntiles, n_exp),
        in_specs=[
            pl.BlockSpec((tile, d_in), lambda t, e: (t, 0)),
            pl.BlockSpec((d_in, n_exp), lambda t, e: (0, 0)),
            pl.BlockSpec((1, d_in, d_hid), lambda t, e: (e, 0, 0)),
            pl.BlockSpec((1, 1, d_hid), lambda t, e: (e, 0, 0)),
            pl.BlockSpec((1, d_hid, d_out), lambda t, e: (e, 0, 0)),
            pl.BlockSpec((1, 1, d_out), lambda t, e: (e, 0, 0)),
        ],
        out_specs=pl.BlockSpec((tile, d_out), lambda t, e: (t, 0)),
        out_shape=jax.ShapeDtypeStruct((n_tok, d_out), x.dtype),
        scratch_shapes=[pltpu.VMEM((tile, 128), jnp.int32)],
    )(x, W_sel, W1, b1r, W2, b2r)
    return out


# speedup vs baseline: 1.9112x; 1.9112x over previous
import jax
import jax.numpy as jnp
from jax import lax
from jax.experimental import pallas as pl
from jax.experimental.pallas import tpu as pltpu
from jax.experimental.pallas import tpu_sc as plsc

N_TOPICS = 8
D_IN = 1024
D_HID = 1024
D_OUT = 3
N_TOK = 8192
DP = 128

NC = 2
NS = 16
L = 16
NW = NC * NS
CHUNK = N_TOK // NW
NGR = CHUNK // L
SCAT = 64
NSCAT = CHUNK // SCAT

T = 256
NWORK = N_TOK // T + (N_TOPICS - 1) + 1
NPAD = NWORK * T
EIDN = 48


def _gate_body(x_ref, wsel_ref, top_ref, hist_ref):
    logits = jnp.dot(x_ref[...], wsel_ref[...],
                     preferred_element_type=jnp.float32)
    best = logits[:, 0:1]
    idx = jnp.zeros((logits.shape[0], 1), jnp.int32)
    for j in range(1, N_TOPICS):
        lj = logits[:, j:j + 1]
        take = lj > best
        best = jnp.where(take, lj, best)
        idx = jnp.where(take, j, idx)
    top_ref[...] = idx
    ids = lax.broadcasted_iota(jnp.int32, (logits.shape[0], L), 1)
    onehot = jnp.where(jnp.broadcast_to(idx, ids.shape) == ids, 1, 0)
    hist_ref[0] = jnp.sum(onehot, axis=0, keepdims=True)


def _gating(x, W_sel):
    out, hist = pl.pallas_call(
        _gate_body,
        grid=(NW,),
        in_specs=[
            pl.BlockSpec((CHUNK, D_IN), lambda t: (t, 0)),
            pl.BlockSpec((D_IN, N_TOPICS), lambda t: (0, 0)),
        ],
        out_specs=[
            pl.BlockSpec((CHUNK, 1), lambda t: (t, 0)),
            pl.BlockSpec((1, 1, L), lambda t: (t, 0, 0)),
        ],
        out_shape=[
            jax.ShapeDtypeStruct((N_TOK, 1), jnp.int32),
            jax.ShapeDtypeStruct((NW, 1, L), jnp.int32),
        ],
    )(x, W_sel)
    return out.reshape(N_TOK), hist.reshape(NW * L)


def _route_body(topics_hbm, hist_hbm, x_hbm, dst_hbm, eid_hbm, xpad_hbm,
                tv, hv, dv, i0, i1, i2, i3, eidv, xbuf, sem):
    wid = lax.axis_index("s") * NC + lax.axis_index("c")
    base = wid * CHUNK
    lane = lax.iota(jnp.int32, L)
    idx_refs = [i0, i1, i2, i3]

    pltpu.sync_copy(topics_hbm.at[pl.ds(base, CHUNK)], tv)
    pltpu.sync_copy(hist_hbm, hv)

    widv = jnp.full((L,), wid, jnp.int32)
    totals = jnp.zeros((L,), jnp.int32)
    before = jnp.zeros((L,), jnp.int32)
    for w in range(NW):
        row = hv[pl.ds(w * L, L)]
        totals = totals + row
        wv = jnp.full((L,), w, jnp.int32)
        before = before + jnp.where(wv < widv, row, 0)
    pc = ((totals + (T - 1)) // T) * T
    pad_off = plsc.cumsum(pc) - pc
    startv = pad_off + before

    tile_end = plsc.cumsum(pc // T)
    for grp in range(EIDN // L):
        j = lane + grp * L
        acc = jnp.zeros((L,), jnp.int32)
        for e in range(N_TOPICS):
            te = jnp.sum(jnp.where(lane == e, tile_end, 0))
            acc = acc + jnp.where(j >= te, 1, 0)
        eidv[pl.ds(grp * L, L)] = jnp.minimum(acc, N_TOPICS - 1)

    @pl.when(wid == 0)
    def _store_eid():
        pltpu.sync_copy(eidv, eid_hbm)

    for g in range(NGR):
        t16 = tv[pl.ds(g * L, L)]
        d16 = jnp.zeros((L,), jnp.int32)
        for e in range(N_TOPICS):
            m = t16 == e
            mi = jnp.where(m, 1, 0)
            csum = plsc.cumsum(mi)
            base_e = jnp.sum(jnp.where(lane == e, startv, 0))
            d16 = jnp.where(m, base_e + csum - 1, d16)
            startv = startv + jnp.where(lane == e, jnp.sum(mi), 0)
        d16 = jnp.clip(d16, 0, NPAD - 1)
        dv[pl.ds(g * L, L)] = d16
        idx_refs[g // (SCAT // L)][pl.ds((g % (SCAT // L)) * L, L)] = d16

    pltpu.sync_copy(dv, dst_hbm.at[pl.ds(base, CHUNK)])

    for r in range(NSCAT):
        pltpu.sync_copy(x_hbm.at[pl.ds(base + r * SCAT, SCAT)], xbuf)
        pltpu.async_copy(xbuf, xpad_hbm.at[idx_refs[r]], sem).wait()


def _route(topics, hist, x):
    mesh = plsc.VectorSubcoreMesh(core_axis_name="c", subcore_axis_name="s")
    fn = pl.kernel(
        _route_body,
        out_type=[
            jax.ShapeDtypeStruct((N_TOK,), jnp.int32),
            jax.ShapeDtypeStruct((EIDN,), jnp.int32),
            jax.ShapeDtypeStruct((NPAD, D_IN), jnp.float32),
        ],
        mesh=mesh,
        scratch_types=[
            pltpu.VMEM((CHUNK,), jnp.int32),
            pltpu.VMEM((NW * L,), jnp.int32),
            pltpu.VMEM((CHUNK,), jnp.int32),
            pltpu.VMEM((SCAT,), jnp.int32),
            pltpu.VMEM((SCAT,), jnp.int32),
            pltpu.VMEM((SCAT,), jnp.int32),
            pltpu.VMEM((SCAT,), jnp.int32),
            pltpu.VMEM((EIDN,), jnp.int32),
            pltpu.VMEM((SCAT, D_IN), jnp.float32),
            pltpu.SemaphoreType.DMA,
        ],
        compiler_params=pltpu.CompilerParams(needs_layout_passes=False),
    )
    return fn(topics, hist, x)


def _mlp_body(eid_ref, x_ref, w1_ref, b1_ref, w2_ref, b2_ref, out_ref):
    h = jnp.maximum(
        jnp.dot(x_ref[...], w1_ref[0], preferred_element_type=jnp.float32)
        + b1_ref[0], 0.0)
    out_ref[...] = jnp.dot(h, w2_ref[0],
                           preferred_element_type=jnp.float32) + b2_ref[0]


def _mlp(eid, x_pad, W1, b1r, W2p, b2p):
    grid_spec = pltpu.PrefetchScalarGridSpec(
        num_scalar_prefetch=1,
        grid=(NWORK,),
        in_specs=[
            pl.BlockSpec((T, D_IN), lambda w, eid: (w, 0)),
            pl.BlockSpec((1, D_IN, D_HID), lambda w, eid: (eid[w], 0, 0)),
            pl.BlockSpec((1, 1, D_HID), lambda w, eid: (eid[w], 0, 0)),
            pl.BlockSpec((1, D_HID, DP), lambda w, eid: (eid[w], 0, 0)),
            pl.BlockSpec((1, 1, DP), lambda w, eid: (eid[w], 0, 0)),
        ],
        out_specs=pl.BlockSpec((T, DP), lambda w, eid: (w, 0)),
    )
    return pl.pallas_call(
        _mlp_body,
        grid_spec=grid_spec,
        out_shape=jax.ShapeDtypeStruct((NPAD, DP), jnp.float32),
    )(eid, x_pad, W1, b1r, W2p, b2p)


def _ungather_body(dst_hbm, pp_hbm, out_hbm, idxv, buf, sem):
    wid = lax.axis_index("s") * NC + lax.axis_index("c")
    base = wid * CHUNK
    pltpu.sync_copy(dst_hbm.at[pl.ds(base, CHUNK)], idxv)
    pltpu.async_copy(pp_hbm.at[idxv], buf, sem).wait()
    pltpu.sync_copy(buf, out_hbm.at[pl.ds(base, CHUNK)])


def _ungather(dst, preds_pad):
    mesh = plsc.VectorSubcoreMesh(core_axis_name="c", subcore_axis_name="s")
    fn = pl.kernel(
        _ungather_body,
        out_type=jax.ShapeDtypeStruct((N_TOK, DP), jnp.float32),
        mesh=mesh,
        scratch_types=[
            pltpu.VMEM((CHUNK,), jnp.int32),
            pltpu.VMEM((CHUNK, DP), jnp.float32),
            pltpu.SemaphoreType.DMA,
        ],
        compiler_params=pltpu.CompilerParams(needs_layout_passes=False),
    )
    return fn(dst, preds_pad)


def kernel(x, W_sel, W1, b1, W2, b2):
    topics, hist = _gating(x, W_sel)
    dst, eid, x_pad = _route(topics, hist, x)
    b1r = b1.reshape(N_TOPICS, 1, D_HID)
    W2p = jnp.pad(W2, ((0, 0), (0, 0), (0, DP - D_OUT)))
    b2p = jnp.pad(b2, ((0, 0), (0, DP - D_OUT))).reshape(N_TOPICS, 1, DP)
    preds_pad = _mlp(eid[:NWORK], x_pad, W1, b1r, W2p, b2p)
    out16 = _ungather(dst, preds_pad)
    return out16[:, :D_OUT]


# speedup vs baseline: 1.9143x; 1.0016x over previous
import jax
import jax.numpy as jnp
from jax import lax
from jax.experimental import pallas as pl
from jax.experimental.pallas import tpu as pltpu
from jax.experimental.pallas import tpu_sc as plsc

N_TOPICS = 8
D_IN = 1024
D_HID = 1024
D_OUT = 3
N_TOK = 8192
DP = 128

NC = 2
NS = 16
L = 16
NW = NC * NS
CHUNK = N_TOK // NW
NGR = CHUNK // L
SCAT = 64
NSCAT = CHUNK // SCAT

T = 256
NWORK = N_TOK // T + (N_TOPICS - 1) + 1
NPAD = NWORK * T
EIDN = 48


def _gate_body(x_ref, wsel_ref, top_ref, hist_ref):
    logits = jnp.dot(x_ref[...], wsel_ref[...],
                     preferred_element_type=jnp.float32)
    best = logits[:, 0:1]
    idx = jnp.zeros((logits.shape[0], 1), jnp.int32)
    for j in range(1, N_TOPICS):
        lj = logits[:, j:j + 1]
        take = lj > best
        best = jnp.where(take, lj, best)
        idx = jnp.where(take, j, idx)
    top_ref[...] = idx
    ids = lax.broadcasted_iota(jnp.int32, (logits.shape[0], L), 1)
    onehot = jnp.where(jnp.broadcast_to(idx, ids.shape) == ids, 1, 0)
    hist_ref[0] = jnp.sum(onehot, axis=0, keepdims=True)


def _gating(x, W_sel):
    out, hist = pl.pallas_call(
        _gate_body,
        grid=(NW,),
        in_specs=[
            pl.BlockSpec((CHUNK, D_IN), lambda t: (t, 0)),
            pl.BlockSpec((D_IN, N_TOPICS), lambda t: (0, 0)),
        ],
        out_specs=[
            pl.BlockSpec((CHUNK, 1), lambda t: (t, 0)),
            pl.BlockSpec((1, 1, L), lambda t: (t, 0, 0)),
        ],
        out_shape=[
            jax.ShapeDtypeStruct((N_TOK, 1), jnp.int32),
            jax.ShapeDtypeStruct((NW, 1, L), jnp.int32),
        ],
    )(x, W_sel)
    return out.reshape(N_TOK), hist.reshape(NW * L)


def _route_body(topics_hbm, hist_hbm, x_hbm, dst_hbm, eid_hbm, xpad_hbm,
                tv, hv, dv, i0, i1, i2, i3, eidv, xbuf, sem):
    wid = lax.axis_index("s") * NC + lax.axis_index("c")
    base = wid * CHUNK
    lane = lax.iota(jnp.int32, L)
    idx_refs = [i0, i1, i2, i3]

    pltpu.sync_copy(topics_hbm.at[pl.ds(base, CHUNK)], tv)
    pltpu.sync_copy(hist_hbm, hv)

    widv = jnp.full((L,), wid, jnp.int32)
    totals = jnp.zeros((L,), jnp.int32)
    before = jnp.zeros((L,), jnp.int32)
    for w in range(NW):
        row = hv[pl.ds(w * L, L)]
        totals = totals + row
        wv = jnp.full((L,), w, jnp.int32)
        before = before + jnp.where(wv < widv, row, 0)
    pc = ((totals + (T - 1)) // T) * T
    pad_off = plsc.cumsum(pc) - pc
    startv = pad_off + before

    tile_end = plsc.cumsum(pc // T)
    for grp in range(EIDN // L):
        j = lane + grp * L
        acc = jnp.zeros((L,), jnp.int32)
        for e in range(N_TOPICS):
            te = jnp.sum(jnp.where(lane == e, tile_end, 0))
            acc = acc + jnp.where(j >= te, 1, 0)
        eidv[pl.ds(grp * L, L)] = jnp.minimum(acc, N_TOPICS - 1)

    @pl.when(wid == 0)
    def _store_eid():
        pltpu.sync_copy(eidv, eid_hbm)

    for g in range(NGR):
        t16 = tv[pl.ds(g * L, L)]
        d16 = jnp.zeros((L,), jnp.int32)
        for e in range(N_TOPICS):
            m = t16 == e
            mi = jnp.where(m, 1, 0)
            csum = plsc.cumsum(mi)
            base_e = jnp.sum(jnp.where(lane == e, startv, 0))
            d16 = jnp.where(m, base_e + csum - 1, d16)
            startv = startv + jnp.where(lane == e, jnp.sum(mi), 0)
        d16 = jnp.clip(d16, 0, NPAD - 1)
        dv[pl.ds(g * L, L)] = d16
        idx_refs[g // (SCAT // L)][pl.ds((g % (SCAT // L)) * L, L)] = d16

    pltpu.sync_copy(dv, dst_hbm.at[pl.ds(base, CHUNK)])

    for r in range(NSCAT):
        pltpu.sync_copy(x_hbm.at[pl.ds(base + r * SCAT, SCAT)], xbuf)
        pltpu.async_copy(xbuf, xpad_hbm.at[idx_refs[r]], sem).wait()


def _route(topics, hist, x):
    mesh = plsc.VectorSubcoreMesh(core_axis_name="c", subcore_axis_name="s")
    fn = pl.kernel(
        _route_body,
        out_type=[
            jax.ShapeDtypeStruct((N_TOK,), jnp.int32),
            jax.ShapeDtypeStruct((EIDN,), jnp.int32),
            jax.ShapeDtypeStruct((NPAD, D_IN), jnp.float32),
        ],
        mesh=mesh,
        scratch_types=[
            pltpu.VMEM((CHUNK,), jnp.int32),
            pltpu.VMEM((NW * L,), jnp.int32),
            pltpu.VMEM((CHUNK,), jnp.int32),
            pltpu.VMEM((SCAT,), jnp.int32),
            pltpu.VMEM((SCAT,), jnp.int32),
            pltpu.VMEM((SCAT,), jnp.int32),
            pltpu.VMEM((SCAT,), jnp.int32),
            pltpu.VMEM((EIDN,), jnp.int32),
            pltpu.VMEM((SCAT, D_IN), jnp.float32),
            pltpu.SemaphoreType.DMA,
        ],
        compiler_params=pltpu.CompilerParams(needs_layout_passes=False),
    )
    return fn(topics, hist, x)


DO8 = 8


def _mlp_body(eid_ref, x_ref, w1_ref, b1_ref, w2_ref, b2_ref, out_ref, w1b):
    w = pl.program_id(0)
    cur = eid_ref[w]
    prev = eid_ref[jnp.maximum(w - 1, 0)]

    @pl.when((w == 0) | (cur != prev))
    def _cast_w1():
        w1b[...] = w1_ref[0].astype(jnp.bfloat16)

    xb = x_ref[...].astype(jnp.bfloat16)
    h = jnp.maximum(
        jnp.dot(xb, w1b[...], preferred_element_type=jnp.float32)
        + b1_ref[0], 0.0)
    o = jnp.dot(h.astype(jnp.bfloat16), w2_ref[0].astype(jnp.bfloat16),
                preferred_element_type=jnp.float32) + b2_ref[0]
    out_ref[:, 0:DO8] = o


def _mlp(eid, x_pad, W1, b1r, W2p, b2p):
    grid_spec = pltpu.PrefetchScalarGridSpec(
        num_scalar_prefetch=1,
        grid=(NWORK,),
        in_specs=[
            pl.BlockSpec((T, D_IN), lambda w, eid: (w, 0)),
            pl.BlockSpec((1, D_IN, D_HID), lambda w, eid: (eid[w], 0, 0)),
            pl.BlockSpec((1, 1, D_HID), lambda w, eid: (eid[w], 0, 0)),
            pl.BlockSpec((1, D_HID, DO8), lambda w, eid: (eid[w], 0, 0)),
            pl.BlockSpec((1, 1, DO8), lambda w, eid: (eid[w], 0, 0)),
        ],
        out_specs=pl.BlockSpec((T, DP), lambda w, eid: (w, 0)),
        scratch_shapes=[pltpu.VMEM((D_IN, D_HID), jnp.bfloat16)],
    )
    return pl.pallas_call(
        _mlp_body,
        grid_spec=grid_spec,
        out_shape=jax.ShapeDtypeStruct((NPAD, DP), jnp.float32),
    )(eid, x_pad, W1, b1r, W2p, b2p)


def _ungather_body(dst_hbm, pp_hbm, out_hbm, idxv, buf, sem):
    wid = lax.axis_index("s") * NC + lax.axis_index("c")
    base = wid * CHUNK
    pltpu.sync_copy(dst_hbm.at[pl.ds(base, CHUNK)], idxv)
    pltpu.async_copy(pp_hbm.at[idxv], buf, sem).wait()
    pltpu.sync_copy(buf, out_hbm.at[pl.ds(base, CHUNK)])


def _ungather(dst, preds_pad):
    mesh = plsc.VectorSubcoreMesh(core_axis_name="c", subcore_axis_name="s")
    fn = pl.kernel(
        _ungather_body,
        out_type=jax.ShapeDtypeStruct((N_TOK, DP), jnp.float32),
        mesh=mesh,
        scratch_types=[
            pltpu.VMEM((CHUNK,), jnp.int32),
            pltpu.VMEM((CHUNK, DP), jnp.float32),
            pltpu.SemaphoreType.DMA,
        ],
        compiler_params=pltpu.CompilerParams(needs_layout_passes=False),
    )
    return fn(dst, preds_pad)


def kernel(x, W_sel, W1, b1, W2, b2):
    topics, hist = _gating(x, W_sel)
    dst, eid, x_pad = _route(topics, hist, x)
    b1r = b1.reshape(N_TOPICS, 1, D_HID)
    W2p = jnp.pad(W2, ((0, 0), (0, 0), (0, DO8 - D_OUT)))
    b2p = jnp.pad(b2, ((0, 0), (0, DO8 - D_OUT))).reshape(N_TOPICS, 1, DO8)
    preds_pad = _mlp(eid[:NWORK], x_pad, W1, b1r, W2p, b2p)
    out16 = _ungather(dst, preds_pad)
    return out16[:, :D_OUT]


# speedup vs baseline: 1.9486x; 1.0179x over previous
import jax
import jax.numpy as jnp
from jax import lax
from jax.experimental import pallas as pl
from jax.experimental.pallas import tpu as pltpu
from jax.experimental.pallas import tpu_sc as plsc

N_TOPICS = 8
D_IN = 1024
D_HID = 1024
D_OUT = 3
N_TOK = 8192
DP = 128

NC = 2
NS = 16
L = 16
NW = NC * NS
CHUNK = N_TOK // NW
NGR = CHUNK // L
SCAT = 32
NSCAT = CHUNK // SCAT

T = 256
NWORK = N_TOK // T + (N_TOPICS - 1) + 1
NPAD = NWORK * T
EIDN = 48


def _gate_body(x_ref, wsel_ref, top_ref, hist_ref):
    logits = jnp.dot(x_ref[...], wsel_ref[...],
                     preferred_element_type=jnp.float32)
    n = logits.shape[0]
    ids8 = lax.broadcasted_iota(jnp.int32, (n, N_TOPICS), 1)
    best = jnp.max(logits, axis=1, keepdims=True)
    idx = jnp.min(jnp.where(logits == best, ids8, N_TOPICS),
                  axis=1, keepdims=True)
    top_ref[...] = idx
    ids = lax.broadcasted_iota(jnp.int32, (n, L), 1)
    onehot = jnp.where(jnp.broadcast_to(idx, ids.shape) == ids, 1, 0)
    hist_ref[0] = jnp.sum(onehot, axis=0, keepdims=True)


def _gating(x, W_sel):
    out, hist = pl.pallas_call(
        _gate_body,
        grid=(NW,),
        in_specs=[
            pl.BlockSpec((CHUNK, D_IN), lambda t: (t, 0)),
            pl.BlockSpec((D_IN, N_TOPICS), lambda t: (0, 0)),
        ],
        out_specs=[
            pl.BlockSpec((CHUNK, 1), lambda t: (t, 0)),
            pl.BlockSpec((1, 1, L), lambda t: (t, 0, 0)),
        ],
        out_shape=[
            jax.ShapeDtypeStruct((N_TOK, 1), jnp.int32),
            jax.ShapeDtypeStruct((NW, 1, L), jnp.int32),
        ],
    )(x, W_sel)
    return out.reshape(N_TOK), hist.reshape(NW * L)


def _route_body(topics_hbm, hist_hbm, x_hbm, dst_hbm, eid_hbm, xpad_hbm,
                tv, hv, dv, i0, i1, i2, i3, i4, i5, i6, i7, eidv,
                xbuf0, xbuf1, lsem, ssem):
    wid = lax.axis_index("s") * NC + lax.axis_index("c")
    base = wid * CHUNK
    lane = lax.iota(jnp.int32, L)
    idx_refs = [i0, i1, i2, i3, i4, i5, i6, i7]
    xbufs = [xbuf0, xbuf1]

    pltpu.sync_copy(topics_hbm.at[pl.ds(base, CHUNK)], tv)
    pltpu.sync_copy(hist_hbm, hv)

    widv = jnp.full((L,), wid, jnp.int32)
    totals = jnp.zeros((L,), jnp.int32)
    before = jnp.zeros((L,), jnp.int32)
    for w in range(NW):
        row = hv[pl.ds(w * L, L)]
        totals = totals + row
        wv = jnp.full((L,), w, jnp.int32)
        before = before + jnp.where(wv < widv, row, 0)
    pc = ((totals + (T - 1)) // T) * T
    pad_off = plsc.cumsum(pc) - pc
    startv = pad_off + before

    tile_end = plsc.cumsum(pc // T)
    for grp in range(EIDN // L):
        j = lane + grp * L
        acc = jnp.zeros((L,), jnp.int32)
        for e in range(N_TOPICS):
            te = jnp.sum(jnp.where(lane == e, tile_end, 0))
            acc = acc + jnp.where(j >= te, 1, 0)
        eidv[pl.ds(grp * L, L)] = jnp.minimum(acc, N_TOPICS - 1)

    @pl.when(wid == 0)
    def _store_eid():
        pltpu.sync_copy(eidv, eid_hbm)

    for g in range(NGR):
        t16 = tv[pl.ds(g * L, L)]
        d16 = jnp.zeros((L,), jnp.int32)
        for e in range(N_TOPICS):
            m = t16 == e
            mi = jnp.where(m, 1, 0)
            csum = plsc.cumsum(mi)
            base_e = jnp.sum(jnp.where(lane == e, startv, 0))
            d16 = jnp.where(m, base_e + csum - 1, d16)
            startv = startv + jnp.where(lane == e, jnp.sum(mi), 0)
        d16 = jnp.clip(d16, 0, NPAD - 1)
        dv[pl.ds(g * L, L)] = d16
        idx_refs[g // (SCAT // L)][pl.ds((g % (SCAT // L)) * L, L)] = d16

    pltpu.sync_copy(dv, dst_hbm.at[pl.ds(base, CHUNK)])

    loads = [pltpu.async_copy(x_hbm.at[pl.ds(base, SCAT)], xbufs[0], lsem)]
    scats = []
    for r in range(NSCAT):
        if r + 1 < NSCAT:
            if r >= 1:
                scats[r - 1].wait()
            loads.append(pltpu.async_copy(
                x_hbm.at[pl.ds(base + (r + 1) * SCAT, SCAT)],
                xbufs[(r + 1) % 2], lsem))
        loads[r].wait()
        scats.append(pltpu.async_copy(xbufs[r % 2],
                                      xpad_hbm.at[idx_refs[r]], ssem))
    scats[NSCAT - 2].wait()
    scats[NSCAT - 1].wait()


def _route(topics, hist, x):
    mesh = plsc.VectorSubcoreMesh(core_axis_name="c", subcore_axis_name="s")
    fn = pl.kernel(
        _route_body,
        out_type=[
            jax.ShapeDtypeStruct((N_TOK,), jnp.int32),
            jax.ShapeDtypeStruct((EIDN,), jnp.int32),
            jax.ShapeDtypeStruct((NPAD, D_IN), jnp.float32),
        ],
        mesh=mesh,
        scratch_types=(
            [
                pltpu.VMEM((CHUNK,), jnp.int32),
                pltpu.VMEM((NW * L,), jnp.int32),
                pltpu.VMEM((CHUNK,), jnp.int32),
            ]
            + [pltpu.VMEM((SCAT,), jnp.int32) for _ in range(NSCAT)]
            + [
                pltpu.VMEM((EIDN,), jnp.int32),
                pltpu.VMEM((SCAT, D_IN), jnp.float32),
                pltpu.VMEM((SCAT, D_IN), jnp.float32),
                pltpu.SemaphoreType.DMA,
                pltpu.SemaphoreType.DMA,
            ]
        ),
        compiler_params=pltpu.CompilerParams(needs_layout_passes=False),
    )
    return fn(topics, hist, x)


DO8 = 8


def _mlp_body(eid_ref, x_ref, w1_ref, b1_ref, w2_ref, b2_ref, out_ref):
    xb = x_ref[...].astype(jnp.bfloat16)
    h = jnp.maximum(
        jnp.dot(xb, w1_ref[0], preferred_element_type=jnp.float32)
        + b1_ref[0], 0.0)
    o = jnp.dot(h.astype(jnp.bfloat16), w2_ref[0],
                preferred_element_type=jnp.float32) + b2_ref[0]
    out_ref[:, 0:DO8] = o


def _mlp(eid, x_pad, W1, b1r, W2p, b2p):
    grid_spec = pltpu.PrefetchScalarGridSpec(
        num_scalar_prefetch=1,
        grid=(NWORK,),
        in_specs=[
            pl.BlockSpec((T, D_IN), lambda w, eid: (w, 0)),
            pl.BlockSpec((1, D_IN, D_HID), lambda w, eid: (eid[w], 0, 0)),
            pl.BlockSpec((1, 1, D_HID), lambda w, eid: (eid[w], 0, 0)),
            pl.BlockSpec((1, D_HID, DO8), lambda w, eid: (eid[w], 0, 0)),
            pl.BlockSpec((1, 1, DO8), lambda w, eid: (eid[w], 0, 0)),
        ],
        out_specs=pl.BlockSpec((T, DP), lambda w, eid: (w, 0)),
    )
    return pl.pallas_call(
        _mlp_body,
        grid_spec=grid_spec,
        out_shape=jax.ShapeDtypeStruct((NPAD, DP), jnp.float32),
    )(eid, x_pad, W1, b1r, W2p, b2p)


def _ungather_body(dst_hbm, pp_hbm, out_hbm, idxv, buf, sem):
    wid = lax.axis_index("s") * NC + lax.axis_index("c")
    base = wid * CHUNK
    pltpu.sync_copy(dst_hbm.at[pl.ds(base, CHUNK)], idxv)
    pltpu.async_copy(pp_hbm.at[idxv], buf, sem).wait()
    pltpu.sync_copy(buf, out_hbm.at[pl.ds(base, CHUNK)])


def _ungather(dst, preds_pad):
    mesh = plsc.VectorSubcoreMesh(core_axis_name="c", subcore_axis_name="s")
    fn = pl.kernel(
        _ungather_body,
        out_type=jax.ShapeDtypeStruct((N_TOK, DP), jnp.float32),
        mesh=mesh,
        scratch_types=[
            pltpu.VMEM((CHUNK,), jnp.int32),
            pltpu.VMEM((CHUNK, DP), jnp.float32),
            pltpu.SemaphoreType.DMA,
        ],
        compiler_params=pltpu.CompilerParams(needs_layout_passes=False),
    )
    return fn(dst, preds_pad)


def kernel(x, W_sel, W1, b1, W2, b2):
    topics, hist = _gating(x, W_sel)
    dst, eid, x_pad = _route(topics, hist, x)
    b1r = b1.reshape(N_TOPICS, 1, D_HID)
    W1b = W1.astype(jnp.bfloat16)
    W2b = jnp.pad(W2, ((0, 0), (0, 0), (0, DO8 - D_OUT))).astype(jnp.bfloat16)
    b2p = jnp.pad(b2, ((0, 0), (0, DO8 - D_OUT))).reshape(N_TOPICS, 1, DO8)
    preds_pad = _mlp(eid[:NWORK], x_pad, W1b, b1r, W2b, b2p)
    out16 = _ungather(dst, preds_pad)
    return out16[:, :D_OUT]


# speedup vs baseline: 2.2308x; 1.1448x over previous
import jax
import jax.numpy as jnp
from jax import lax
from jax.experimental import pallas as pl
from jax.experimental.pallas import tpu as pltpu
from jax.experimental.pallas import tpu_sc as plsc

N_TOPICS = 8
D_IN = 1024
D_HID = 1024
D_OUT = 3
N_TOK = 8192
DP = 128

NC = 2
NS = 16
L = 16
NW = NC * NS
CHUNK = N_TOK // NW
NGR = CHUNK // L
SCAT = 64
NSCAT = CHUNK // SCAT

T = 256
NWORK = N_TOK // T + (N_TOPICS - 1) + 1
NPAD = NWORK * T
EIDN = 48


GT = 1024
GSUB = GT // CHUNK


def _gate_body(x_ref, wsel_ref, top_ref, hist_ref):
    xv = x_ref[...]
    logits = jnp.dot(xv, wsel_ref[...], preferred_element_type=jnp.float32)
    n = logits.shape[0]
    ids8 = lax.broadcasted_iota(jnp.int32, (n, N_TOPICS), 1)
    best = jnp.max(logits, axis=1, keepdims=True)
    idx = jnp.min(jnp.where(logits == best, ids8, N_TOPICS),
                  axis=1, keepdims=True)
    top_ref[...] = idx
    ids = lax.broadcasted_iota(jnp.int32, (n, L), 1)
    onehot = jnp.where(jnp.broadcast_to(idx, ids.shape) == ids, 1, 0)
    for s in range(GSUB):
        hist_ref[0, s:s + 1, :] = jnp.sum(
            onehot[s * CHUNK:(s + 1) * CHUNK], axis=0, keepdims=True)


def _gating(x, W_sel):
    out, hist = pl.pallas_call(
        _gate_body,
        grid=(N_TOK // GT,),
        in_specs=[
            pl.BlockSpec((GT, D_IN), lambda t: (t, 0)),
            pl.BlockSpec((D_IN, N_TOPICS), lambda t: (0, 0)),
        ],
        out_specs=[
            pl.BlockSpec((GT, 1), lambda t: (t, 0)),
            pl.BlockSpec((1, GSUB, L), lambda t: (t, 0, 0)),
        ],
        out_shape=[
            jax.ShapeDtypeStruct((N_TOK, 1), jnp.int32),
            jax.ShapeDtypeStruct((N_TOK // GT, GSUB, L), jnp.int32),
        ],
    )(x, W_sel)
    return out.reshape(N_TOK), hist.reshape(NW * L)


def _route_body(topics_hbm, hist_hbm, x_hbm, dst_hbm, eid_hbm, xpad_hbm,
                tv, hv, dv, i0, i1, i2, i3, eidv,
                xbuf0, xbuf1, lsem, ssem):
    wid = lax.axis_index("s") * NC + lax.axis_index("c")
    base = wid * CHUNK
    lane = lax.iota(jnp.int32, L)
    idx_refs = [i0, i1, i2, i3]
    xbufs = [xbuf0, xbuf1]

    pltpu.sync_copy(topics_hbm.at[pl.ds(base, CHUNK)], tv)
    pltpu.sync_copy(hist_hbm, hv)

    widv = jnp.full((L,), wid, jnp.int32)
    totals = jnp.zeros((L,), jnp.int32)
    before = jnp.zeros((L,), jnp.int32)
    for w in range(NW):
        row = hv[pl.ds(w * L, L)]
        totals = totals + row
        wv = jnp.full((L,), w, jnp.int32)
        before = before + jnp.where(wv < widv, row, 0)
    pc = ((totals + (T - 1)) // T) * T
    pad_off = plsc.cumsum(pc) - pc
    startv = pad_off + before

    tile_end = plsc.cumsum(pc // T)
    for grp in range(EIDN // L):
        j = lane + grp * L
        acc = jnp.zeros((L,), jnp.int32)
        for e in range(N_TOPICS):
            te = jnp.sum(jnp.where(lane == e, tile_end, 0))
            acc = acc + jnp.where(j >= te, 1, 0)
        eidv[pl.ds(grp * L, L)] = jnp.minimum(acc, N_TOPICS - 1)

    @pl.when(wid == 0)
    def _store_eid():
        pltpu.sync_copy(eidv, eid_hbm)

    for g in range(NGR):
        t16 = tv[pl.ds(g * L, L)]
        d16 = jnp.zeros((L,), jnp.int32)
        for e in range(N_TOPICS):
            m = t16 == e
            mi = jnp.where(m, 1, 0)
            csum = plsc.cumsum(mi)
            base_e = jnp.sum(jnp.where(lane == e, startv, 0))
            d16 = jnp.where(m, base_e + csum - 1, d16)
            startv = startv + jnp.where(lane == e, jnp.sum(mi), 0)
        d16 = jnp.clip(d16, 0, NPAD - 1)
        dv[pl.ds(g * L, L)] = d16
        idx_refs[g // (SCAT // L)][pl.ds((g % (SCAT // L)) * L, L)] = d16

    pltpu.sync_copy(dv, dst_hbm.at[pl.ds(base, CHUNK)])

    for r in range(NSCAT):
        pltpu.sync_copy(x_hbm.at[pl.ds(base + r * SCAT, SCAT)], xbufs[0])
        pltpu.async_copy(xbufs[0], xpad_hbm.at[idx_refs[r]], ssem).wait()


def _route(topics, hist, x):
    mesh = plsc.VectorSubcoreMesh(core_axis_name="c", subcore_axis_name="s")
    fn = pl.kernel(
        _route_body,
        out_type=[
            jax.ShapeDtypeStruct((N_TOK,), jnp.int32),
            jax.ShapeDtypeStruct((EIDN,), jnp.int32),
            jax.ShapeDtypeStruct((NPAD, D_IN), jnp.float32),
        ],
        mesh=mesh,
        scratch_types=(
            [
                pltpu.VMEM((CHUNK,), jnp.int32),
                pltpu.VMEM((NW * L,), jnp.int32),
                pltpu.VMEM((CHUNK,), jnp.int32),
            ]
            + [pltpu.VMEM((SCAT,), jnp.int32) for _ in range(NSCAT)]
            + [
                pltpu.VMEM((EIDN,), jnp.int32),
                pltpu.VMEM((SCAT, D_IN), jnp.float32),
                pltpu.VMEM((SCAT // 2, D_IN), jnp.float32),
                pltpu.SemaphoreType.DMA,
                pltpu.SemaphoreType.DMA,
            ]
        ),
        compiler_params=pltpu.CompilerParams(needs_layout_passes=False),
    )
    return fn(topics, hist, x)


DO8 = 8


def _mlp_body(eid_ref, x_ref, w1_ref, b1_ref, w2_ref, b2_ref, out_ref, w1b):
    w = pl.program_id(0)
    cur = eid_ref[w]
    prev = eid_ref[jnp.maximum(w - 1, 0)]

    @pl.when((w == 0) | (cur != prev))
    def _cast_w1():
        w1b[...] = w1_ref[0].astype(jnp.bfloat16)

    xb = x_ref[...].astype(jnp.bfloat16)
    h = jnp.maximum(
        jnp.dot(xb, w1b[...], preferred_element_type=jnp.float32)
        + b1_ref[0], 0.0)
    o = jnp.dot(h.astype(jnp.bfloat16), w2_ref[0],
                preferred_element_type=jnp.float32) + b2_ref[0]
    out_ref[:, 0:DO8] = o


def _mlp(eid, x_pad, W1, b1r, W2p, b2p):
    grid_spec = pltpu.PrefetchScalarGridSpec(
        num_scalar_prefetch=1,
        grid=(NWORK,),
        in_specs=[
            pl.BlockSpec((T, D_IN), lambda w, eid: (w, 0)),
            pl.BlockSpec((1, D_IN, D_HID), lambda w, eid: (eid[w], 0, 0)),
            pl.BlockSpec((1, 1, D_HID), lambda w, eid: (eid[w], 0, 0)),
            pl.BlockSpec((1, D_HID, DO8), lambda w, eid: (eid[w], 0, 0)),
            pl.BlockSpec((1, 1, DO8), lambda w, eid: (eid[w], 0, 0)),
        ],
        out_specs=pl.BlockSpec((T, DP), lambda w, eid: (w, 0)),
        scratch_shapes=[pltpu.VMEM((D_IN, D_HID), jnp.bfloat16)],
    )
    return pl.pallas_call(
        _mlp_body,
        grid_spec=grid_spec,
        out_shape=jax.ShapeDtypeStruct((NPAD, DP), jnp.float32),
    )(eid, x_pad, W1, b1r, W2p, b2p)


def _ungather_body(dst_hbm, pp_hbm, out_hbm, idxv, buf, sem):
    wid = lax.axis_index("s") * NC + lax.axis_index("c")
    base = wid * CHUNK
    pltpu.sync_copy(dst_hbm.at[pl.ds(base, CHUNK)], idxv)
    pltpu.async_copy(pp_hbm.at[idxv], buf, sem).wait()
    pltpu.sync_copy(buf, out_hbm.at[pl.ds(base, CHUNK)])


def _ungather(dst, preds_pad):
    mesh = plsc.VectorSubcoreMesh(core_axis_name="c", subcore_axis_name="s")
    fn = pl.kernel(
        _ungather_body,
        out_type=jax.ShapeDtypeStruct((N_TOK, DP), jnp.float32),
        mesh=mesh,
        scratch_types=[
            pltpu.VMEM((CHUNK,), jnp.int32),
            pltpu.VMEM((CHUNK, DP), jnp.float32),
            pltpu.SemaphoreType.DMA,
        ],
        compiler_params=pltpu.CompilerParams(needs_layout_passes=False),
    )
    return fn(dst, preds_pad)


def kernel(x, W_sel, W1, b1, W2, b2):
    topics, hist = _gating(x, W_sel)
    dst, eid, x_pad = _route(topics, hist, x)
    b1r = b1.reshape(N_TOPICS, 1, D_HID)
    W2b = jnp.pad(W2, ((0, 0), (0, 0), (0, DO8 - D_OUT))).astype(jnp.bfloat16)
    b2p = jnp.pad(b2, ((0, 0), (0, DO8 - D_OUT))).reshape(N_TOPICS, 1, DO8)
    preds_pad = _mlp(eid[:NWORK], x_pad, W1, b1r, W2b, b2p)
    out16 = _ungather(dst, preds_pad)
    return out16[:, :D_OUT]
